# baseline (speedup 1.0000x reference)
import functools

import jax
import jax.numpy as jnp
from jax import lax
from jax.experimental import pallas as pl
from jax.experimental.pallas import tpu as pltpu
from jax.experimental.pallas import tpu_sc as plsc

B, T, V, D, H, NC = 1024, 50, 100000, 100, 128, 7
DP = 128
G4 = 4 * H
BT = B * T

_NCORE, _NSUB = 2, 16
_NW = _NCORE * _NSUB
_PER_W = BT // _NW
_CHUNK = 80
_NCHUNK = _PER_W // _CHUNK


def _sc_gather(table, idx_flat):
  mesh = plsc.VectorSubcoreMesh(core_axis_name="c", subcore_axis_name="s")

  @functools.partial(
      pl.kernel,
      out_type=jax.ShapeDtypeStruct((BT, DP), jnp.float32),
      mesh=mesh,
      scratch_types=[
          pltpu.VMEM((_CHUNK,), jnp.int32),
          pltpu.VMEM((_CHUNK, DP), jnp.float32),
          pltpu.SemaphoreType.DMA,
      ],
  )
  def gather_kernel(table_hbm, idx_hbm, out_hbm, idx_v, rows_v, sem):
    wid = lax.axis_index("s") * _NCORE + lax.axis_index("c")
    base = wid * _PER_W

    def body(j, carry):
      off = base + j * _CHUNK
      pltpu.sync_copy(idx_hbm.at[pl.ds(off, _CHUNK)], idx_v)
      pltpu.async_copy(table_hbm.at[idx_v], rows_v, sem).wait()
      pltpu.sync_copy(rows_v, out_hbm.at[pl.ds(off, _CHUNK)])
      return carry

    lax.fori_loop(0, _NCHUNK, body, 0)

  return gather_kernel(table, idx_flat)


def _tpad_body(xt_ref, o_ref):
  blk = o_ref.shape[0]
  rows = xt_ref[...].T
  o_ref[...] = jnp.concatenate(
      [rows, jnp.zeros((blk, DP - D), jnp.float32)], axis=1)


def _pad_table(table_t):
  blkc = 2048
  nblk = (V + blkc - 1) // blkc
  return pl.pallas_call(
      _tpad_body,
      grid=(nblk,),
      in_specs=[pl.BlockSpec((D, blkc), lambda i: (0, i))],
      out_specs=pl.BlockSpec((blkc, DP), lambda i: (i, 0)),
      out_shape=jax.ShapeDtypeStruct((V, DP), jnp.float32),
  )(table_t)


def _lstm_step(x_parts, w_parts, whh_t, bias, h, c):
  g = bias[...]
  for xp, wp in zip(x_parts, w_parts):
    g = g + jnp.dot(xp.astype(jnp.bfloat16), wp,
                    preferred_element_type=jnp.float32)
  g = g + jnp.dot(h[...].astype(jnp.bfloat16), whh_t[...],
                  preferred_element_type=jnp.float32)
  gi = jax.nn.sigmoid(g[:, :H])
  gf = jax.nn.sigmoid(g[:, H:2 * H])
  gg = jnp.tanh(g[:, 2 * H:3 * H])
  go = jax.nn.sigmoid(g[:, 3 * H:])
  c2 = gf * c[...] + gi * gg
  h2 = go * jnp.tanh(c2)
  return h2, c2


def _layer0_body(xf_ref, xb_ref, wif, whf, bsf, wib, whb, bsb,
                 yf_ref, yb_ref, hf, cf, hb, cb):
  t = pl.program_id(0)

  @pl.when(t == 0)
  def _():
    for r in (hf, cf, hb, cb):
      r[...] = jnp.zeros_like(r)

  h2f, c2f = _lstm_step([xf_ref[...]], [wif[...]], whf, bsf, hf, cf)
  hf[...] = h2f
  cf[...] = c2f
  yf_ref[...] = h2f.astype(jnp.bfloat16)

  h2b, c2b = _lstm_step([xb_ref[...]], [wib[...]], whb, bsb, hb, cb)
  hb[...] = h2b
  cb[...] = c2b
  yb_ref[...] = h2b.astype(jnp.bfloat16)


def _layer1_body(ff_ref, fb_ref, rf_ref, rb_ref, wif, whf, bsf, wib, whb, bsb,
                 fcw, fcb, out_ref, hf, cf, hb, cb, yb_last):
  t = pl.program_id(0)

  @pl.when(t == 0)
  def _():
    for r in (hf, cf, hb, cb):
      r[...] = jnp.zeros_like(r)

  h2f, c2f = _lstm_step([ff_ref[...], fb_ref[...]],
                        [wif[:H, :], wif[H:, :]], whf, bsf, hf, cf)
  hf[...] = h2f
  cf[...] = c2f

  h2b, c2b = _lstm_step([rf_ref[...], rb_ref[...]],
                        [wib[:H, :], wib[H:, :]], whb, bsb, hb, cb)
  hb[...] = h2b
  cb[...] = c2b

  @pl.when(t == 0)
  def _():
    yb_last[...] = h2b

  @pl.when(t == T - 1)
  def _():
    logits = (jnp.dot(h2f, fcw[:H, :], preferred_element_type=jnp.float32)
              + jnp.dot(yb_last[...], fcw[H:, :],
                        preferred_element_type=jnp.float32)
              + fcb[...])
    out_ref[...] = logits


def _rep(shape):
  return pl.BlockSpec(shape, lambda t: tuple(0 for _ in shape))


def _bilstm_l0(x, wif_t, whf_t, bsf, wib_t, whb_t, bsb):
  fwd = pl.BlockSpec((B, DP), lambda t: (t, 0))
  rev = pl.BlockSpec((B, DP), lambda t: (T - 1 - t, 0))
  return pl.pallas_call(
      _layer0_body,
      grid=(T,),
      in_specs=[fwd, rev, _rep((DP, G4)), _rep((H, G4)), _rep((1, G4)),
                _rep((DP, G4)), _rep((H, G4)), _rep((1, G4))],
      out_specs=[pl.BlockSpec((B, H), lambda t: (t, 0)),
                 pl.BlockSpec((B, H), lambda t: (T - 1 - t, 0))],
      out_shape=[jax.ShapeDtypeStruct((BT, H), jnp.bfloat16)] * 2,
      scratch_shapes=[pltpu.VMEM((B, H), jnp.float32)] * 4,
  )(x, x, wif_t, whf_t, bsf, wib_t, whb_t, bsb)


def _bilstm_l1_fc(yf, yb, wif_t, whf_t, bsf, wib_t, whb_t, bsb, fcw_t, fcb):
  fwd = pl.BlockSpec((B, H), lambda t: (t, 0))
  rev = pl.BlockSpec((B, H), lambda t: (T - 1 - t, 0))
  return pl.pallas_call(
      _layer1_body,
      grid=(T,),
      in_specs=[fwd, fwd, rev, rev,
                _rep((2 * H, G4)), _rep((H, G4)), _rep((1, G4)),
                _rep((2 * H, G4)), _rep((H, G4)), _rep((1, G4)),
                _rep((2 * H, NC)), _rep((1, NC))],
      out_specs=pl.BlockSpec((B, NC), lambda t: (0, 0)),
      out_shape=jax.ShapeDtypeStruct((B, NC), jnp.float32),
      scratch_shapes=[pltpu.VMEM((B, H), jnp.float32)] * 5,
  )(yf, yb, yf, yb, wif_t, whf_t, bsf, wib_t, whb_t, bsb, fcw_t, fcb)


def kernel(indices, emb_table,
           W_ih_l0_f, W_hh_l0_f, b_ih_l0_f, b_hh_l0_f,
           W_ih_l0_b, W_hh_l0_b, b_ih_l0_b, b_hh_l0_b,
           W_ih_l1_f, W_hh_l1_f, b_ih_l1_f, b_hh_l1_f,
           W_ih_l1_b, W_hh_l1_b, b_ih_l1_b, b_hh_l1_b,
           fc_W, fc_b):
  idx_flat = indices.T.reshape(BT)
  table_p = _pad_table(emb_table.T)
  x = _sc_gather(table_p, idx_flat)

  def prep(wih, whh, bih, bhh, pad=0):
    wt = wih.T
    if pad:
      wt = jnp.pad(wt, ((0, pad), (0, 0)))
    return (wt.astype(jnp.bfloat16), whh.T.astype(jnp.bfloat16),
            (bih + bhh).reshape(1, G4))

  w0f = prep(W_ih_l0_f, W_hh_l0_f, b_ih_l0_f, b_hh_l0_f, DP - D)
  w0b = prep(W_ih_l0_b, W_hh_l0_b, b_ih_l0_b, b_hh_l0_b, DP - D)
  yf, ybk = _bilstm_l0(x, *w0f, *w0b)

  w1f = prep(W_ih_l1_f, W_hh_l1_f, b_ih_l1_f, b_hh_l1_f)
  w1b = prep(W_ih_l1_b, W_hh_l1_b, b_ih_l1_b, b_hh_l1_b)
  return _bilstm_l1_fc(yf, ybk, *w1f, *w1b, fc_W.T, fc_b.reshape(1, NC))


# speedup vs baseline: 1.0674x; 1.0674x over previous
import functools

import jax
import jax.numpy as jnp
from jax import lax
from jax.experimental import pallas as pl
from jax.experimental.pallas import tpu as pltpu
from jax.experimental.pallas import tpu_sc as plsc

B, T, V, D, H, NC = 1024, 50, 100000, 100, 128, 7
DP = 128
G4 = 4 * H
BT = B * T

_NCORE, _NSUB = 2, 16
_NW = _NCORE * _NSUB
_PER_W = BT // _NW
_CHUNK = 80
_NCHUNK = _PER_W // _CHUNK


def _sc_gather(table, idx_flat):
  mesh = plsc.VectorSubcoreMesh(core_axis_name="c", subcore_axis_name="s")

  @functools.partial(
      pl.kernel,
      out_type=jax.ShapeDtypeStruct((BT, DP), jnp.float32),
      mesh=mesh,
      scratch_types=[
          pltpu.VMEM((_CHUNK,), jnp.int32),
          pltpu.VMEM((_CHUNK, DP), jnp.float32),
          pltpu.SemaphoreType.DMA,
      ],
  )
  def gather_kernel(table_hbm, idx_hbm, out_hbm, idx_v, rows_v, sem):
    wid = lax.axis_index("s") * _NCORE + lax.axis_index("c")
    base = wid * _PER_W

    def body(j, carry):
      off = base + j * _CHUNK
      pltpu.sync_copy(idx_hbm.at[pl.ds(off, _CHUNK)], idx_v)
      pltpu.async_copy(table_hbm.at[idx_v], rows_v, sem).wait()
      pltpu.sync_copy(rows_v, out_hbm.at[pl.ds(off, _CHUNK)])
      return carry

    lax.fori_loop(0, _NCHUNK, body, 0)

  return gather_kernel(table, idx_flat)


def _tpad_body(xt_ref, o_ref):
  blk = o_ref.shape[0]
  rows = xt_ref[...].T
  o_ref[...] = jnp.concatenate(
      [rows, jnp.zeros((blk, DP - D), jnp.float32)], axis=1)


def _pad_table(table_t):
  blkc = 2048
  nblk = (V + blkc - 1) // blkc
  return pl.pallas_call(
      _tpad_body,
      grid=(nblk,),
      in_specs=[pl.BlockSpec((D, blkc), lambda i: (0, i))],
      out_specs=pl.BlockSpec((blkc, DP), lambda i: (i, 0)),
      out_shape=jax.ShapeDtypeStruct((V, DP), jnp.float32),
  )(table_t)


def _lstm_step(x_parts, w_parts, whh_t, bias, h, c):
  g = bias[...]
  for xp, wp in zip(x_parts, w_parts):
    g = g + jnp.dot(xp.astype(jnp.bfloat16), wp,
                    preferred_element_type=jnp.float32)
  g = g + jnp.dot(h[...].astype(jnp.bfloat16), whh_t[...],
                  preferred_element_type=jnp.float32)
  def sig(x):
    return 0.5 * jnp.tanh(0.5 * x) + 0.5
  gi = sig(g[:, :H])
  gf = sig(g[:, H:2 * H])
  gg = jnp.tanh(g[:, 2 * H:3 * H])
  go = sig(g[:, 3 * H:])
  c2 = gf * c[...] + gi * gg
  h2 = go * jnp.tanh(c2)
  return h2, c2


def _layer0_body(xf_ref, xb_ref, wif, whf, bsf, wib, whb, bsb,
                 yf_ref, yb_ref, hf, cf, hb, cb):
  t = pl.program_id(0)

  @pl.when(t == 0)
  def _():
    for r in (hf, cf, hb, cb):
      r[...] = jnp.zeros_like(r)

  h2f, c2f = _lstm_step([xf_ref[...]], [wif[...]], whf, bsf, hf, cf)
  hf[...] = h2f
  cf[...] = c2f
  yf_ref[...] = h2f.astype(jnp.bfloat16)

  h2b, c2b = _lstm_step([xb_ref[...]], [wib[...]], whb, bsb, hb, cb)
  hb[...] = h2b
  cb[...] = c2b
  yb_ref[...] = h2b.astype(jnp.bfloat16)


def _layer1_body(ff_ref, fb_ref, rf_ref, rb_ref, wif, whf, bsf, wib, whb, bsb,
                 fcw, fcb, out_ref, hf, cf, hb, cb, yb_last):
  t = pl.program_id(0)

  @pl.when(t == 0)
  def _():
    for r in (hf, cf, hb, cb):
      r[...] = jnp.zeros_like(r)

  h2f, c2f = _lstm_step([ff_ref[...], fb_ref[...]],
                        [wif[:H, :], wif[H:, :]], whf, bsf, hf, cf)
  hf[...] = h2f
  cf[...] = c2f

  h2b, c2b = _lstm_step([rf_ref[...], rb_ref[...]],
                        [wib[:H, :], wib[H:, :]], whb, bsb, hb, cb)
  hb[...] = h2b
  cb[...] = c2b

  @pl.when(t == 0)
  def _():
    yb_last[...] = h2b

  @pl.when(t == T - 1)
  def _():
    logits = (jnp.dot(h2f, fcw[:H, :], preferred_element_type=jnp.float32)
              + jnp.dot(yb_last[...], fcw[H:, :],
                        preferred_element_type=jnp.float32)
              + fcb[...])
    out_ref[...] = logits


def _rep(shape):
  return pl.BlockSpec(shape, lambda t: tuple(0 for _ in shape))


def _bilstm_l0(x, wif_t, whf_t, bsf, wib_t, whb_t, bsb):
  fwd = pl.BlockSpec((B, DP), lambda t: (t, 0))
  rev = pl.BlockSpec((B, DP), lambda t: (T - 1 - t, 0))
  return pl.pallas_call(
      _layer0_body,
      grid=(T,),
      in_specs=[fwd, rev, _rep((DP, G4)), _rep((H, G4)), _rep((1, G4)),
                _rep((DP, G4)), _rep((H, G4)), _rep((1, G4))],
      out_specs=[pl.BlockSpec((B, H), lambda t: (t, 0)),
                 pl.BlockSpec((B, H), lambda t: (T - 1 - t, 0))],
      out_shape=[jax.ShapeDtypeStruct((BT, H), jnp.bfloat16)] * 2,
      scratch_shapes=[pltpu.VMEM((B, H), jnp.float32)] * 4,
  )(x, x, wif_t, whf_t, bsf, wib_t, whb_t, bsb)


def _bilstm_l1_fc(yf, yb, wif_t, whf_t, bsf, wib_t, whb_t, bsb, fcw_t, fcb):
  fwd = pl.BlockSpec((B, H), lambda t: (t, 0))
  rev = pl.BlockSpec((B, H), lambda t: (T - 1 - t, 0))
  return pl.pallas_call(
      _layer1_body,
      grid=(T,),
      in_specs=[fwd, fwd, rev, rev,
                _rep((2 * H, G4)), _rep((H, G4)), _rep((1, G4)),
                _rep((2 * H, G4)), _rep((H, G4)), _rep((1, G4)),
                _rep((2 * H, NC)), _rep((1, NC))],
      out_specs=pl.BlockSpec((B, NC), lambda t: (0, 0)),
      out_shape=jax.ShapeDtypeStruct((B, NC), jnp.float32),
      scratch_shapes=[pltpu.VMEM((B, H), jnp.float32)] * 5,
  )(yf, yb, yf, yb, wif_t, whf_t, bsf, wib_t, whb_t, bsb, fcw_t, fcb)


def kernel(indices, emb_table,
           W_ih_l0_f, W_hh_l0_f, b_ih_l0_f, b_hh_l0_f,
           W_ih_l0_b, W_hh_l0_b, b_ih_l0_b, b_hh_l0_b,
           W_ih_l1_f, W_hh_l1_f, b_ih_l1_f, b_hh_l1_f,
           W_ih_l1_b, W_hh_l1_b, b_ih_l1_b, b_hh_l1_b,
           fc_W, fc_b):
  idx_flat = indices.T.reshape(BT)
  table_p = _pad_table(emb_table.T)
  x = _sc_gather(table_p, idx_flat)

  def prep(wih, whh, bih, bhh, pad=0):
    wt = wih.T
    if pad:
      wt = jnp.pad(wt, ((0, pad), (0, 0)))
    return (wt.astype(jnp.bfloat16), whh.T.astype(jnp.bfloat16),
            (bih + bhh).reshape(1, G4))

  w0f = prep(W_ih_l0_f, W_hh_l0_f, b_ih_l0_f, b_hh_l0_f, DP - D)
  w0b = prep(W_ih_l0_b, W_hh_l0_b, b_ih_l0_b, b_hh_l0_b, DP - D)
  yf, ybk = _bilstm_l0(x, *w0f, *w0b)

  w1f = prep(W_ih_l1_f, W_hh_l1_f, b_ih_l1_f, b_hh_l1_f)
  w1b = prep(W_ih_l1_b, W_hh_l1_b, b_ih_l1_b, b_hh_l1_b)
  return _bilstm_l1_fc(yf, ybk, *w1f, *w1b, fc_W.T, fc_b.reshape(1, NC))


# speedup vs baseline: 1.2503x; 1.1714x over previous
import functools

import jax
import jax.numpy as jnp
from jax import lax
from jax.experimental import pallas as pl
from jax.experimental.pallas import tpu as pltpu
from jax.experimental.pallas import tpu_sc as plsc

B, T, V, D, H, NC = 1024, 50, 100000, 100, 128, 7
DP = 128
G4 = 4 * H
BT = B * T

_NCORE, _NSUB = 2, 16
_NW = _NCORE * _NSUB
_PER_W = BT // _NW
_CHUNK = 80
_NCHUNK = _PER_W // _CHUNK


def _sc_gather(table, idx_flat):
  mesh = plsc.VectorSubcoreMesh(core_axis_name="c", subcore_axis_name="s")

  @functools.partial(
      pl.kernel,
      out_type=jax.ShapeDtypeStruct((BT, DP), jnp.float32),
      mesh=mesh,
      scratch_types=[
          pltpu.VMEM((_CHUNK,), jnp.int32),
          pltpu.VMEM((_CHUNK, DP), jnp.float32),
          pltpu.SemaphoreType.DMA,
      ],
  )
  def gather_kernel(table_hbm, idx_hbm, out_hbm, idx_v, rows_v, sem):
    wid = lax.axis_index("s") * _NCORE + lax.axis_index("c")
    base = wid * _PER_W

    def body(j, carry):
      off = base + j * _CHUNK
      pltpu.sync_copy(idx_hbm.at[pl.ds(off, _CHUNK)], idx_v)
      pltpu.async_copy(table_hbm.at[idx_v], rows_v, sem).wait()
      pltpu.sync_copy(rows_v, out_hbm.at[pl.ds(off, _CHUNK)])
      return carry

    lax.fori_loop(0, _NCHUNK, body, 0)

  return gather_kernel(table, idx_flat)


def _tpad_body(xt_ref, o_ref):
  blk = o_ref.shape[0]
  rows = xt_ref[...].T
  o_ref[...] = jnp.concatenate(
      [rows, jnp.zeros((blk, DP - D), jnp.float32)], axis=1)


def _pad_table(table_t):
  blkc = 2048
  nblk = (V + blkc - 1) // blkc
  return pl.pallas_call(
      _tpad_body,
      grid=(nblk,),
      in_specs=[pl.BlockSpec((D, blkc), lambda i: (0, i))],
      out_specs=pl.BlockSpec((blkc, DP), lambda i: (i, 0)),
      out_shape=jax.ShapeDtypeStruct((V, DP), jnp.float32),
  )(table_t)


def _lstm_step(x_parts, w_t, bias, h, c):
  xin = jnp.concatenate(
      [xp.astype(jnp.bfloat16) for xp in x_parts]
      + [h[...].astype(jnp.bfloat16)], axis=1)
  g = jnp.dot(xin, w_t[...], preferred_element_type=jnp.float32) + bias[...]
  def sig(x):
    return 0.5 * jnp.tanh(0.5 * x) + 0.5
  gi = sig(g[:, :H])
  gf = sig(g[:, H:2 * H])
  gg = jnp.tanh(g[:, 2 * H:3 * H])
  go = sig(g[:, 3 * H:])
  c2 = gf * c[...] + gi * gg
  h2 = go * jnp.tanh(c2)
  return h2, c2


def _layer0_body(xf_ref, xb_ref, wf, bsf, wb, bsb,
                 yf_ref, yb_ref, hf, cf, hb, cb):
  t = pl.program_id(0)

  @pl.when(t == 0)
  def _():
    for r in (hf, cf, hb, cb):
      r[...] = jnp.zeros_like(r)

  h2f, c2f = _lstm_step([xf_ref[...]], wf, bsf, hf, cf)
  hf[...] = h2f
  cf[...] = c2f
  yf_ref[...] = h2f.astype(jnp.bfloat16)

  h2b, c2b = _lstm_step([xb_ref[...]], wb, bsb, hb, cb)
  hb[...] = h2b
  cb[...] = c2b
  yb_ref[...] = h2b.astype(jnp.bfloat16)


def _layer1_body(ff_ref, fb_ref, rf_ref, rb_ref, wf, bsf, wb, bsb,
                 fcw, fcb, out_ref, hf, cf, hb, cb, yb_last):
  t = pl.program_id(0)

  @pl.when(t == 0)
  def _():
    for r in (hf, cf, hb, cb):
      r[...] = jnp.zeros_like(r)

  h2f, c2f = _lstm_step([ff_ref[...], fb_ref[...]], wf, bsf, hf, cf)
  hf[...] = h2f
  cf[...] = c2f

  h2b, c2b = _lstm_step([rf_ref[...], rb_ref[...]], wb, bsb, hb, cb)
  hb[...] = h2b
  cb[...] = c2b

  @pl.when(t == 0)
  def _():
    yb_last[...] = h2b

  @pl.when(t == T - 1)
  def _():
    logits = (jnp.dot(h2f, fcw[:H, :], preferred_element_type=jnp.float32)
              + jnp.dot(yb_last[...], fcw[H:, :],
                        preferred_element_type=jnp.float32)
              + fcb[...])
    out_ref[...] = logits


def _rep(shape):
  return pl.BlockSpec(shape, lambda t: tuple(0 for _ in shape))


def _bilstm_l0(x, wf, bsf, wb, bsb):
  fwd = pl.BlockSpec((B, DP), lambda t: (t, 0))
  rev = pl.BlockSpec((B, DP), lambda t: (T - 1 - t, 0))
  return pl.pallas_call(
      _layer0_body,
      grid=(T,),
      in_specs=[fwd, rev, _rep((DP + H, G4)), _rep((1, G4)),
                _rep((DP + H, G4)), _rep((1, G4))],
      out_specs=[pl.BlockSpec((B, H), lambda t: (t, 0)),
                 pl.BlockSpec((B, H), lambda t: (T - 1 - t, 0))],
      out_shape=[jax.ShapeDtypeStruct((BT, H), jnp.bfloat16)] * 2,
      scratch_shapes=[pltpu.VMEM((B, H), jnp.float32)] * 4,
  )(x, x, wf, bsf, wb, bsb)


def _bilstm_l1_fc(yf, yb, wf, bsf, wb, bsb, fcw_t, fcb):
  fwd = pl.BlockSpec((B, H), lambda t: (t, 0))
  rev = pl.BlockSpec((B, H), lambda t: (T - 1 - t, 0))
  return pl.pallas_call(
      _layer1_body,
      grid=(T,),
      in_specs=[fwd, fwd, rev, rev,
                _rep((3 * H, G4)), _rep((1, G4)),
                _rep((3 * H, G4)), _rep((1, G4)),
                _rep((2 * H, NC)), _rep((1, NC))],
      out_specs=pl.BlockSpec((B, NC), lambda t: (0, 0)),
      out_shape=jax.ShapeDtypeStruct((B, NC), jnp.float32),
      scratch_shapes=[pltpu.VMEM((B, H), jnp.float32)] * 5,
  )(yf, yb, yf, yb, wf, bsf, wb, bsb, fcw_t, fcb)


def kernel(indices, emb_table,
           W_ih_l0_f, W_hh_l0_f, b_ih_l0_f, b_hh_l0_f,
           W_ih_l0_b, W_hh_l0_b, b_ih_l0_b, b_hh_l0_b,
           W_ih_l1_f, W_hh_l1_f, b_ih_l1_f, b_hh_l1_f,
           W_ih_l1_b, W_hh_l1_b, b_ih_l1_b, b_hh_l1_b,
           fc_W, fc_b):
  idx_flat = indices.T.reshape(BT)
  table_p = _pad_table(emb_table.T)
  x = _sc_gather(table_p, idx_flat)

  def prep(wih, whh, bih, bhh, pad=0):
    wt = wih.T
    if pad:
      wt = jnp.pad(wt, ((0, pad), (0, 0)))
    wcat = jnp.concatenate([wt, whh.T], axis=0).astype(jnp.bfloat16)
    return wcat, (bih + bhh).reshape(1, G4)

  w0f = prep(W_ih_l0_f, W_hh_l0_f, b_ih_l0_f, b_hh_l0_f, DP - D)
  w0b = prep(W_ih_l0_b, W_hh_l0_b, b_ih_l0_b, b_hh_l0_b, DP - D)
  yf, ybk = _bilstm_l0(x, *w0f, *w0b)

  w1f = prep(W_ih_l1_f, W_hh_l1_f, b_ih_l1_f, b_hh_l1_f)
  w1b = prep(W_ih_l1_b, W_hh_l1_b, b_ih_l1_b, b_hh_l1_b)
  return _bilstm_l1_fc(yf, ybk, *w1f, *w1b, fc_W.T, fc_b.reshape(1, NC))


# speedup vs baseline: 1.3967x; 1.1171x over previous
import functools

import jax
import jax.numpy as jnp
from jax import lax
from jax.experimental import pallas as pl
from jax.experimental.pallas import tpu as pltpu
from jax.experimental.pallas import tpu_sc as plsc

B, T, V, D, H, NC = 1024, 50, 100000, 100, 128, 7
DP = 128
G4 = 4 * H
BT = B * T

_NCORE, _NSUB = 2, 16
_NW = _NCORE * _NSUB
_PER_W = BT // _NW
_CHUNK = 80
_NCHUNK = _PER_W // _CHUNK


def _sc_gather(table, idx_flat):
  mesh = plsc.VectorSubcoreMesh(core_axis_name="c", subcore_axis_name="s")

  @functools.partial(
      pl.kernel,
      out_type=jax.ShapeDtypeStruct((BT, DP), jnp.float32),
      mesh=mesh,
      scratch_types=[
          pltpu.VMEM((_PER_W,), jnp.int32),
          pltpu.VMEM((_CHUNK, DP), jnp.float32),
          pltpu.VMEM((_CHUNK, DP), jnp.float32),
          pltpu.SemaphoreType.DMA,
          pltpu.SemaphoreType.DMA,
      ],
  )
  def gather_kernel(table_hbm, idx_hbm, out_hbm, idx_v, rows_a, rows_b, sem_a,
                    sem_b):
    wid = lax.axis_index("s") * _NCORE + lax.axis_index("c")
    base = wid * _PER_W
    pltpu.sync_copy(idx_hbm.at[pl.ds(base, _PER_W)], idx_v)
    bufs = ((rows_a, sem_a), (rows_b, sem_b))

    def start(j):
      rows, sem = bufs[j % 2]
      return pltpu.async_copy(
          table_hbm.at[idx_v.at[pl.ds(j * _CHUNK, _CHUNK)]], rows, sem)

    pending = start(0)
    for j in range(_NCHUNK):
      nxt = start(j + 1) if j + 1 < _NCHUNK else None
      pending.wait()
      pltpu.sync_copy(bufs[j % 2][0],
                      out_hbm.at[pl.ds(base + j * _CHUNK, _CHUNK)])
      pending = nxt

  return gather_kernel(table, idx_flat)


def _tpad_body(xt_ref, o_ref):
  blk = o_ref.shape[0]
  rows = xt_ref[...].T
  o_ref[...] = jnp.concatenate(
      [rows, jnp.zeros((blk, DP - D), jnp.float32)], axis=1)


def _pad_table(table_t):
  blkc = 2048
  nblk = (V + blkc - 1) // blkc
  return pl.pallas_call(
      _tpad_body,
      grid=(nblk,),
      in_specs=[pl.BlockSpec((D, blkc), lambda i: (0, i))],
      out_specs=pl.BlockSpec((blkc, DP), lambda i: (i, 0)),
      out_shape=jax.ShapeDtypeStruct((V, DP), jnp.float32),
  )(table_t)


def _lstm_step(x_parts, w_t, bias, h, c):
  xin = jnp.concatenate(
      [xp.astype(jnp.bfloat16) for xp in x_parts]
      + [h[...].astype(jnp.bfloat16)], axis=1)
  g = jnp.dot(xin, w_t[...], preferred_element_type=jnp.float32) + bias[...]
  gi = 0.5 * jnp.tanh(g[:, :H]) + 0.5
  gf = 0.5 * jnp.tanh(g[:, H:2 * H]) + 0.5
  gg = jnp.tanh(g[:, 2 * H:3 * H])
  go = 0.5 * jnp.tanh(g[:, 3 * H:]) + 0.5
  c2 = gf * c[...] + gi * gg
  h2 = go * jnp.tanh(c2)
  return h2, c2


def _layer0_body(xf_ref, xb_ref, wf, bsf, wb, bsb,
                 yf_ref, yb_ref, hf, cf, hb, cb):
  t = pl.program_id(0)

  @pl.when(t == 0)
  def _():
    for r in (hf, cf, hb, cb):
      r[...] = jnp.zeros_like(r)

  h2f, c2f = _lstm_step([xf_ref[...]], wf, bsf, hf, cf)
  hf[...] = h2f
  cf[...] = c2f
  yf_ref[...] = h2f.astype(jnp.bfloat16)

  h2b, c2b = _lstm_step([xb_ref[...]], wb, bsb, hb, cb)
  hb[...] = h2b
  cb[...] = c2b
  yb_ref[...] = h2b.astype(jnp.bfloat16)


def _layer1_body(ff_ref, fb_ref, rf_ref, rb_ref, wf, bsf, wb, bsb,
                 fcw, fcb, out_ref, hf, cf, hb, cb, yb_last):
  t = pl.program_id(0)

  @pl.when(t == 0)
  def _():
    for r in (hf, cf, hb, cb):
      r[...] = jnp.zeros_like(r)

  h2f, c2f = _lstm_step([ff_ref[...], fb_ref[...]], wf, bsf, hf, cf)
  hf[...] = h2f
  cf[...] = c2f

  h2b, c2b = _lstm_step([rf_ref[...], rb_ref[...]], wb, bsb, hb, cb)
  hb[...] = h2b
  cb[...] = c2b

  @pl.when(t == 0)
  def _():
    yb_last[...] = h2b

  @pl.when(t == T - 1)
  def _():
    logits = (jnp.dot(h2f, fcw[:H, :], preferred_element_type=jnp.float32)
              + jnp.dot(yb_last[...], fcw[H:, :],
                        preferred_element_type=jnp.float32)
              + fcb[...])
    out_ref[...] = logits


def _rep(shape):
  return pl.BlockSpec(shape, lambda t: tuple(0 for _ in shape))


def _bilstm_l0(x, wf, bsf, wb, bsb):
  fwd = pl.BlockSpec((B, DP), lambda t: (t, 0))
  rev = pl.BlockSpec((B, DP), lambda t: (T - 1 - t, 0))
  return pl.pallas_call(
      _layer0_body,
      grid=(T,),
      in_specs=[fwd, rev, _rep((DP + H, G4)), _rep((1, G4)),
                _rep((DP + H, G4)), _rep((1, G4))],
      out_specs=[pl.BlockSpec((B, H), lambda t: (t, 0)),
                 pl.BlockSpec((B, H), lambda t: (T - 1 - t, 0))],
      out_shape=[jax.ShapeDtypeStruct((BT, H), jnp.bfloat16)] * 2,
      scratch_shapes=[pltpu.VMEM((B, H), jnp.float32)] * 4,
  )(x, x, wf, bsf, wb, bsb)


def _bilstm_l1_fc(yf, yb, wf, bsf, wb, bsb, fcw_t, fcb):
  fwd = pl.BlockSpec((B, H), lambda t: (t, 0))
  rev = pl.BlockSpec((B, H), lambda t: (T - 1 - t, 0))
  return pl.pallas_call(
      _layer1_body,
      grid=(T,),
      in_specs=[fwd, fwd, rev, rev,
                _rep((3 * H, G4)), _rep((1, G4)),
                _rep((3 * H, G4)), _rep((1, G4)),
                _rep((2 * H, NC)), _rep((1, NC))],
      out_specs=pl.BlockSpec((B, NC), lambda t: (0, 0)),
      out_shape=jax.ShapeDtypeStruct((B, NC), jnp.float32),
      scratch_shapes=[pltpu.VMEM((B, H), jnp.float32)] * 5,
  )(yf, yb, yf, yb, wf, bsf, wb, bsb, fcw_t, fcb)


def kernel(indices, emb_table,
           W_ih_l0_f, W_hh_l0_f, b_ih_l0_f, b_hh_l0_f,
           W_ih_l0_b, W_hh_l0_b, b_ih_l0_b, b_hh_l0_b,
           W_ih_l1_f, W_hh_l1_f, b_ih_l1_f, b_hh_l1_f,
           W_ih_l1_b, W_hh_l1_b, b_ih_l1_b, b_hh_l1_b,
           fc_W, fc_b):
  idx_flat = indices.T.reshape(BT)
  table_p = _pad_table(emb_table.T)
  x = _sc_gather(table_p, idx_flat)

  def prep(wih, whh, bih, bhh, pad=0):
    wt = wih.T
    if pad:
      wt = jnp.pad(wt, ((0, pad), (0, 0)))
    wcat = jnp.concatenate([wt, whh.T], axis=0)
    scale = jnp.concatenate([jnp.full((H,), 0.5, jnp.float32)] * 2
                            + [jnp.ones((H,), jnp.float32),
                               jnp.full((H,), 0.5, jnp.float32)])
    return ((wcat * scale[None, :]).astype(jnp.bfloat16),
            ((bih + bhh) * scale).reshape(1, G4))

  w0f = prep(W_ih_l0_f, W_hh_l0_f, b_ih_l0_f, b_hh_l0_f, DP - D)
  w0b = prep(W_ih_l0_b, W_hh_l0_b, b_ih_l0_b, b_hh_l0_b, DP - D)
  yf, ybk = _bilstm_l0(x, *w0f, *w0b)

  w1f = prep(W_ih_l1_f, W_hh_l1_f, b_ih_l1_f, b_hh_l1_f)
  w1b = prep(W_ih_l1_b, W_hh_l1_b, b_ih_l1_b, b_hh_l1_b)
  return _bilstm_l1_fc(yf, ybk, *w1f, *w1b, fc_W.T, fc_b.reshape(1, NC))


# speedup vs baseline: 1.4907x; 1.0673x over previous
import functools

import jax
import jax.numpy as jnp
from jax import lax
from jax.experimental import pallas as pl
from jax.experimental.pallas import tpu as pltpu
from jax.experimental.pallas import tpu_sc as plsc

B, T, V, D, H, NC = 1024, 50, 100000, 100, 128, 7
DP = 128
G4 = 4 * H
BT = B * T

_NCORE, _NSUB = 2, 16
_NW = _NCORE * _NSUB
_PER_W = BT // _NW
_CHUNK = 80
_NCHUNK = _PER_W // _CHUNK


def _sc_gather(table, idx_flat):
  mesh = plsc.VectorSubcoreMesh(core_axis_name="c", subcore_axis_name="s")

  @functools.partial(
      pl.kernel,
      out_type=jax.ShapeDtypeStruct((BT, DP), jnp.float32),
      mesh=mesh,
      scratch_types=[
          pltpu.VMEM((_PER_W,), jnp.int32),
          pltpu.VMEM((_CHUNK, DP), jnp.float32),
          pltpu.VMEM((_CHUNK, DP), jnp.float32),
          pltpu.SemaphoreType.DMA,
          pltpu.SemaphoreType.DMA,
      ],
  )
  def gather_kernel(table_hbm, idx_hbm, out_hbm, idx_v, rows_a, rows_b, sem_a,
                    sem_b):
    wid = lax.axis_index("s") * _NCORE + lax.axis_index("c")
    base = wid * _PER_W
    pltpu.sync_copy(idx_hbm.at[pl.ds(base, _PER_W)], idx_v)
    bufs = ((rows_a, sem_a), (rows_b, sem_b))

    def start(j):
      rows, sem = bufs[j % 2]
      return pltpu.async_copy(
          table_hbm.at[idx_v.at[pl.ds(j * _CHUNK, _CHUNK)]], rows, sem)

    pending = start(0)
    for j in range(_NCHUNK):
      nxt = start(j + 1) if j + 1 < _NCHUNK else None
      pending.wait()
      pltpu.sync_copy(bufs[j % 2][0],
                      out_hbm.at[pl.ds(base + j * _CHUNK, _CHUNK)])
      pending = nxt

  return gather_kernel(table, idx_flat)


def _tpad_body(xt_ref, o_ref):
  blk = o_ref.shape[0]
  rows = xt_ref[...].T
  o_ref[...] = jnp.concatenate(
      [rows, jnp.zeros((blk, DP - D), jnp.float32)], axis=1)


def _pad_table(table_t):
  blkc = 4096
  nblk = (V + blkc - 1) // blkc
  return pl.pallas_call(
      _tpad_body,
      grid=(nblk,),
      in_specs=[pl.BlockSpec((D, blkc), lambda i: (0, i))],
      out_specs=pl.BlockSpec((blkc, DP), lambda i: (i, 0)),
      out_shape=jax.ShapeDtypeStruct((V, DP), jnp.float32),
  )(table_t)


def _lstm_step(x_parts, w_t, bias, h, c):
  xin = jnp.concatenate(
      [xp.astype(jnp.bfloat16) for xp in x_parts]
      + [h[...].astype(jnp.bfloat16)], axis=1)
  g = jnp.dot(xin, w_t[...], preferred_element_type=jnp.float32) + bias[...]
  gi = 0.5 * jnp.tanh(g[:, :H]) + 0.5
  gf = 0.5 * jnp.tanh(g[:, H:2 * H]) + 0.5
  gg = jnp.tanh(g[:, 2 * H:3 * H])
  go = 0.5 * jnp.tanh(g[:, 3 * H:]) + 0.5
  c2 = gf * c[...] + gi * gg
  h2 = go * jnp.tanh(c2)
  return h2, c2


def _layer0_body(xf_ref, xb_ref, wf, bsf, wb, bsb,
                 yf_ref, yb_ref, hf, cf, hb, cb):
  t = pl.program_id(0)

  @pl.when(t == 0)
  def _():
    for r in (hf, cf, hb, cb):
      r[...] = jnp.zeros_like(r)

  h2f, c2f = _lstm_step([xf_ref[...]], wf, bsf, hf, cf)
  hf[...] = h2f
  cf[...] = c2f
  yf_ref[...] = h2f.astype(jnp.bfloat16)

  h2b, c2b = _lstm_step([xb_ref[...]], wb, bsb, hb, cb)
  hb[...] = h2b
  cb[...] = c2b
  yb_ref[...] = h2b.astype(jnp.bfloat16)


def _layer1_body(ff_ref, fb_ref, rf_ref, rb_ref, wf, bsf, wb, bsb,
                 fcw, fcb, out_ref, hf, cf, hb, cb, yb_last):
  t = pl.program_id(0)

  @pl.when(t == 0)
  def _():
    for r in (hf, cf, hb, cb):
      r[...] = jnp.zeros_like(r)

  h2f, c2f = _lstm_step([ff_ref[...], fb_ref[...]], wf, bsf, hf, cf)
  hf[...] = h2f
  cf[...] = c2f

  h2b, c2b = _lstm_step([rf_ref[...], rb_ref[...]], wb, bsb, hb, cb)
  hb[...] = h2b
  cb[...] = c2b

  @pl.when(t == 0)
  def _():
    yb_last[...] = h2b

  @pl.when(t == T - 1)
  def _():
    logits = (jnp.dot(h2f, fcw[:H, :], preferred_element_type=jnp.float32)
              + jnp.dot(yb_last[...], fcw[H:, :],
                        preferred_element_type=jnp.float32)
              + fcb[...])
    out_ref[...] = logits


def _rep(shape):
  return pl.BlockSpec(shape, lambda t: tuple(0 for _ in shape))


def _bilstm_l0(x, wf, bsf, wb, bsb):
  fwd = pl.BlockSpec((B, DP), lambda t: (t, 0))
  rev = pl.BlockSpec((B, DP), lambda t: (T - 1 - t, 0))
  return pl.pallas_call(
      _layer0_body,
      grid=(T,),
      in_specs=[fwd, rev, _rep((DP + H, G4)), _rep((1, G4)),
                _rep((DP + H, G4)), _rep((1, G4))],
      out_specs=[pl.BlockSpec((B, H), lambda t: (t, 0)),
                 pl.BlockSpec((B, H), lambda t: (T - 1 - t, 0))],
      out_shape=[jax.ShapeDtypeStruct((BT, H), jnp.bfloat16)] * 2,
      scratch_shapes=[pltpu.VMEM((B, H), jnp.float32)] * 4,
  )(x, x, wf, bsf, wb, bsb)


def _bilstm_l1_fc(yf, yb, wf, bsf, wb, bsb, fcw_t, fcb):
  fwd = pl.BlockSpec((B, H), lambda t: (t, 0))
  rev = pl.BlockSpec((B, H), lambda t: (T - 1 - t, 0))
  return pl.pallas_call(
      _layer1_body,
      grid=(T,),
      in_specs=[fwd, fwd, rev, rev,
                _rep((3 * H, G4)), _rep((1, G4)),
                _rep((3 * H, G4)), _rep((1, G4)),
                _rep((2 * H, NC)), _rep((1, NC))],
      out_specs=pl.BlockSpec((B, NC), lambda t: (0, 0)),
      out_shape=jax.ShapeDtypeStruct((B, NC), jnp.float32),
      scratch_shapes=[pltpu.VMEM((B, H), jnp.float32)] * 5,
  )(yf, yb, yf, yb, wf, bsf, wb, bsb, fcw_t, fcb)


def kernel(indices, emb_table,
           W_ih_l0_f, W_hh_l0_f, b_ih_l0_f, b_hh_l0_f,
           W_ih_l0_b, W_hh_l0_b, b_ih_l0_b, b_hh_l0_b,
           W_ih_l1_f, W_hh_l1_f, b_ih_l1_f, b_hh_l1_f,
           W_ih_l1_b, W_hh_l1_b, b_ih_l1_b, b_hh_l1_b,
           fc_W, fc_b):
  idx_flat = indices.T.reshape(BT)
  table_p = _pad_table(emb_table.T)
  x = _sc_gather(table_p, idx_flat)

  def prep(wih, whh, bih, bhh, pad=0):
    wt = wih.T
    if pad:
      wt = jnp.pad(wt, ((0, pad), (0, 0)))
    wcat = jnp.concatenate([wt, whh.T], axis=0)
    scale = jnp.concatenate([jnp.full((H,), 0.5, jnp.float32)] * 2
                            + [jnp.ones((H,), jnp.float32),
                               jnp.full((H,), 0.5, jnp.float32)])
    return ((wcat * scale[None, :]).astype(jnp.bfloat16),
            ((bih + bhh) * scale).reshape(1, G4))

  w0f = prep(W_ih_l0_f, W_hh_l0_f, b_ih_l0_f, b_hh_l0_f, DP - D)
  w0b = prep(W_ih_l0_b, W_hh_l0_b, b_ih_l0_b, b_hh_l0_b, DP - D)
  yf, ybk = _bilstm_l0(x, *w0f, *w0b)

  w1f = prep(W_ih_l1_f, W_hh_l1_f, b_ih_l1_f, b_hh_l1_f)
  w1b = prep(W_ih_l1_b, W_hh_l1_b, b_ih_l1_b, b_hh_l1_b)
  return _bilstm_l1_fc(yf, ybk, *w1f, *w1b, fc_W.T, fc_b.reshape(1, NC))


# speedup vs baseline: 1.5331x; 1.0285x over previous
import functools

import jax
import jax.numpy as jnp
from jax import lax
from jax.experimental import pallas as pl
from jax.experimental.pallas import tpu as pltpu
from jax.experimental.pallas import tpu_sc as plsc

B, T, V, D, H, NC = 1024, 50, 100000, 100, 128, 7
DP = 128
G4 = 4 * H
BT = B * T

_NCORE, _NSUB = 2, 16
_NW = _NCORE * _NSUB
_PER_W = BT // _NW
_CHUNK = 80
_NCHUNK = _PER_W // _CHUNK


def _sc_gather(table, idx_flat):
  mesh = plsc.VectorSubcoreMesh(core_axis_name="c", subcore_axis_name="s")

  @functools.partial(
      pl.kernel,
      out_type=jax.ShapeDtypeStruct((BT, DP), jnp.float32),
      mesh=mesh,
      scratch_types=[
          pltpu.VMEM((_PER_W,), jnp.int32),
          pltpu.VMEM((_CHUNK, DP), jnp.float32),
          pltpu.VMEM((_CHUNK, DP), jnp.float32),
          pltpu.SemaphoreType.DMA,
          pltpu.SemaphoreType.DMA,
      ],
  )
  def gather_kernel(table_hbm, idx_hbm, out_hbm, idx_v, rows_a, rows_b, sem_a,
                    sem_b):
    wid = lax.axis_index("s") * _NCORE + lax.axis_index("c")
    base = wid * _PER_W
    pltpu.sync_copy(idx_hbm.at[pl.ds(base, _PER_W)], idx_v)
    bufs = ((rows_a, sem_a), (rows_b, sem_b))

    def start(j):
      rows, sem = bufs[j % 2]
      return pltpu.async_copy(
          table_hbm.at[idx_v.at[pl.ds(j * _CHUNK, _CHUNK)]], rows, sem)

    pending = start(0)
    for j in range(_NCHUNK):
      nxt = start(j + 1) if j + 1 < _NCHUNK else None
      pending.wait()
      pltpu.sync_copy(bufs[j % 2][0],
                      out_hbm.at[pl.ds(base + j * _CHUNK, _CHUNK)])
      pending = nxt

  return gather_kernel(table, idx_flat)


def _tpad_body(xt_ref, o_ref):
  blk = o_ref.shape[0]
  rows = xt_ref[...].T
  o_ref[...] = jnp.concatenate(
      [rows, jnp.zeros((blk, DP - D), jnp.float32)], axis=1)


def _pad_table(table_t):
  blkc = 8192
  nblk = (V + blkc - 1) // blkc
  return pl.pallas_call(
      _tpad_body,
      grid=(nblk,),
      in_specs=[pl.BlockSpec((D, blkc), lambda i: (0, i))],
      out_specs=pl.BlockSpec((blkc, DP), lambda i: (i, 0)),
      out_shape=jax.ShapeDtypeStruct((V, DP), jnp.float32),
  )(table_t)


def _lstm_step(x_parts, w_t, bias, h, c):
  xin = jnp.concatenate(
      [xp.astype(jnp.bfloat16) for xp in x_parts]
      + [h[...].astype(jnp.bfloat16)], axis=1)
  g = jnp.dot(xin, w_t[...], preferred_element_type=jnp.float32) + bias[...]
  gi = 0.5 * jnp.tanh(g[:, :H]) + 0.5
  gf = 0.5 * jnp.tanh(g[:, H:2 * H]) + 0.5
  gg = jnp.tanh(g[:, 2 * H:3 * H])
  go = 0.5 * jnp.tanh(g[:, 3 * H:]) + 0.5
  c2 = gf * c[...] + gi * gg
  h2 = go * jnp.tanh(c2)
  return h2, c2


def _layer0_body(xf_ref, xb_ref, wf, bsf, wb, bsb,
                 yf_ref, yb_ref, hf, cf, hb, cb):
  t = pl.program_id(0)

  @pl.when(t == 0)
  def _():
    for r in (hf, cf, hb, cb):
      r[...] = jnp.zeros_like(r)

  h2f, c2f = _lstm_step([xf_ref[...]], wf, bsf, hf, cf)
  hf[...] = h2f
  cf[...] = c2f
  yf_ref[...] = h2f.astype(jnp.bfloat16)

  h2b, c2b = _lstm_step([xb_ref[...]], wb, bsb, hb, cb)
  hb[...] = h2b
  cb[...] = c2b
  yb_ref[...] = h2b.astype(jnp.bfloat16)


def _layer1_body(ff_ref, fb_ref, rf_ref, rb_ref, wf, bsf, wb, bsb,
                 fcw, fcb, out_ref, hf, cf, hb, cb, yb_last):
  t = pl.program_id(0)

  @pl.when(t == 0)
  def _():
    for r in (hf, cf, hb, cb):
      r[...] = jnp.zeros_like(r)

  h2f, c2f = _lstm_step([ff_ref[...], fb_ref[...]], wf, bsf, hf, cf)
  hf[...] = h2f
  cf[...] = c2f

  h2b, c2b = _lstm_step([rf_ref[...], rb_ref[...]], wb, bsb, hb, cb)
  hb[...] = h2b
  cb[...] = c2b

  @pl.when(t == 0)
  def _():
    yb_last[...] = h2b

  @pl.when(t == T - 1)
  def _():
    logits = (jnp.dot(h2f, fcw[:H, :], preferred_element_type=jnp.float32)
              + jnp.dot(yb_last[...], fcw[H:, :],
                        preferred_element_type=jnp.float32)
              + fcb[...])
    out_ref[...] = logits


def _rep(shape):
  return pl.BlockSpec(shape, lambda t: tuple(0 for _ in shape))


def _bilstm_l0(x, wf, bsf, wb, bsb):
  fwd = pl.BlockSpec((B, DP), lambda t: (t, 0))
  rev = pl.BlockSpec((B, DP), lambda t: (T - 1 - t, 0))
  return pl.pallas_call(
      _layer0_body,
      grid=(T,),
      in_specs=[fwd, rev, _rep((DP + H, G4)), _rep((1, G4)),
                _rep((DP + H, G4)), _rep((1, G4))],
      out_specs=[pl.BlockSpec((B, H), lambda t: (t, 0)),
                 pl.BlockSpec((B, H), lambda t: (T - 1 - t, 0))],
      out_shape=[jax.ShapeDtypeStruct((BT, H), jnp.bfloat16)] * 2,
      scratch_shapes=[pltpu.VMEM((B, H), jnp.float32)] * 4,
  )(x, x, wf, bsf, wb, bsb)


def _bilstm_l1_fc(yf, yb, wf, bsf, wb, bsb, fcw_t, fcb):
  fwd = pl.BlockSpec((B, H), lambda t: (t, 0))
  rev = pl.BlockSpec((B, H), lambda t: (T - 1 - t, 0))
  return pl.pallas_call(
      _layer1_body,
      grid=(T,),
      in_specs=[fwd, fwd, rev, rev,
                _rep((3 * H, G4)), _rep((1, G4)),
                _rep((3 * H, G4)), _rep((1, G4)),
                _rep((2 * H, NC)), _rep((1, NC))],
      out_specs=pl.BlockSpec((B, NC), lambda t: (0, 0)),
      out_shape=jax.ShapeDtypeStruct((B, NC), jnp.float32),
      scratch_shapes=[pltpu.VMEM((B, H), jnp.float32)] * 5,
  )(yf, yb, yf, yb, wf, bsf, wb, bsb, fcw_t, fcb)


def kernel(indices, emb_table,
           W_ih_l0_f, W_hh_l0_f, b_ih_l0_f, b_hh_l0_f,
           W_ih_l0_b, W_hh_l0_b, b_ih_l0_b, b_hh_l0_b,
           W_ih_l1_f, W_hh_l1_f, b_ih_l1_f, b_hh_l1_f,
           W_ih_l1_b, W_hh_l1_b, b_ih_l1_b, b_hh_l1_b,
           fc_W, fc_b):
  idx_flat = indices.T.reshape(BT)
  table_p = _pad_table(emb_table.T)
  x = _sc_gather(table_p, idx_flat)

  def prep(wih, whh, bih, bhh, pad=0):
    wt = wih.T
    if pad:
      wt = jnp.pad(wt, ((0, pad), (0, 0)))
    wcat = jnp.concatenate([wt, whh.T], axis=0)
    scale = jnp.concatenate([jnp.full((H,), 0.5, jnp.float32)] * 2
                            + [jnp.ones((H,), jnp.float32),
                               jnp.full((H,), 0.5, jnp.float32)])
    return ((wcat * scale[None, :]).astype(jnp.bfloat16),
            ((bih + bhh) * scale).reshape(1, G4))

  w0f = prep(W_ih_l0_f, W_hh_l0_f, b_ih_l0_f, b_hh_l0_f, DP - D)
  w0b = prep(W_ih_l0_b, W_hh_l0_b, b_ih_l0_b, b_hh_l0_b, DP - D)
  yf, ybk = _bilstm_l0(x, *w0f, *w0b)

  w1f = prep(W_ih_l1_f, W_hh_l1_f, b_ih_l1_f, b_hh_l1_f)
  w1b = prep(W_ih_l1_b, W_hh_l1_b, b_ih_l1_b, b_hh_l1_b)
  return _bilstm_l1_fc(yf, ybk, *w1f, *w1b, fc_W.T, fc_b.reshape(1, NC))


# speedup vs baseline: 1.5393x; 1.0040x over previous
import functools

import jax
import jax.numpy as jnp
from jax import lax
from jax.experimental import pallas as pl
from jax.experimental.pallas import tpu as pltpu
from jax.experimental.pallas import tpu_sc as plsc

B, T, V, D, H, NC = 1024, 50, 100000, 100, 128, 7
DP = 128
G4 = 4 * H
BT = B * T

_NCORE, _NSUB = 2, 16
_NW = _NCORE * _NSUB
_PER_W = BT // _NW
_CHUNK = 80
_NCHUNK = _PER_W // _CHUNK


def _sc_gather(table, idx_flat):
  mesh = plsc.VectorSubcoreMesh(core_axis_name="c", subcore_axis_name="s")

  @functools.partial(
      pl.kernel,
      out_type=jax.ShapeDtypeStruct((BT, DP), jnp.float32),
      mesh=mesh,
      scratch_types=[
          pltpu.VMEM((_PER_W,), jnp.int32),
          pltpu.VMEM((_CHUNK, DP), jnp.float32),
          pltpu.VMEM((_CHUNK, DP), jnp.float32),
          pltpu.SemaphoreType.DMA,
          pltpu.SemaphoreType.DMA,
      ],
  )
  def gather_kernel(table_hbm, idx_hbm, out_hbm, idx_v, rows_a, rows_b, sem_a,
                    sem_b):
    wid = lax.axis_index("s") * _NCORE + lax.axis_index("c")
    base = wid * _PER_W
    pltpu.sync_copy(idx_hbm.at[pl.ds(base, _PER_W)], idx_v)
    bufs = ((rows_a, sem_a), (rows_b, sem_b))

    def start(j):
      rows, sem = bufs[j % 2]
      return pltpu.async_copy(
          table_hbm.at[idx_v.at[pl.ds(j * _CHUNK, _CHUNK)]], rows, sem)

    pending = start(0)
    for j in range(_NCHUNK):
      nxt = start(j + 1) if j + 1 < _NCHUNK else None
      pending.wait()
      pltpu.sync_copy(bufs[j % 2][0],
                      out_hbm.at[pl.ds(base + j * _CHUNK, _CHUNK)])
      pending = nxt

  return gather_kernel(table, idx_flat)


def _tpad_body(xt_ref, o_ref):
  blk = o_ref.shape[0]
  rows = xt_ref[...].T
  o_ref[...] = jnp.concatenate(
      [rows, jnp.zeros((blk, DP - D), jnp.float32)], axis=1)


def _pad_table(table_t):
  blkc = 16384
  nblk = (V + blkc - 1) // blkc
  return pl.pallas_call(
      _tpad_body,
      grid=(nblk,),
      in_specs=[pl.BlockSpec((D, blkc), lambda i: (0, i))],
      out_specs=pl.BlockSpec((blkc, DP), lambda i: (i, 0)),
      out_shape=jax.ShapeDtypeStruct((V, DP), jnp.float32),
  )(table_t)


def _lstm_step(x_parts, w_t, bias, h, c):
  xin = jnp.concatenate(
      [xp.astype(jnp.bfloat16) for xp in x_parts]
      + [h[...].astype(jnp.bfloat16)], axis=1)
  g = jnp.dot(xin, w_t[...], preferred_element_type=jnp.float32) + bias[...]
  gi = 0.5 * jnp.tanh(g[:, :H]) + 0.5
  gf = 0.5 * jnp.tanh(g[:, H:2 * H]) + 0.5
  gg = jnp.tanh(g[:, 2 * H:3 * H])
  go = 0.5 * jnp.tanh(g[:, 3 * H:]) + 0.5
  c2 = gf * c[...] + gi * gg
  h2 = go * jnp.tanh(c2)
  return h2, c2


def _layer0_body(xf_ref, xb_ref, wf, bsf, wb, bsb,
                 yf_ref, yb_ref, hf, cf, hb, cb):
  t = pl.program_id(0)

  @pl.when(t == 0)
  def _():
    for r in (hf, cf, hb, cb):
      r[...] = jnp.zeros_like(r)

  h2f, c2f = _lstm_step([xf_ref[...]], wf, bsf, hf, cf)
  hf[...] = h2f
  cf[...] = c2f
  yf_ref[...] = h2f.astype(jnp.bfloat16)

  h2b, c2b = _lstm_step([xb_ref[...]], wb, bsb, hb, cb)
  hb[...] = h2b
  cb[...] = c2b
  yb_ref[...] = h2b.astype(jnp.bfloat16)


def _layer1_body(ff_ref, fb_ref, rf_ref, rb_ref, wf, bsf, wb, bsb,
                 fcw, fcb, out_ref, hf, cf, hb, cb, yb_last):
  t = pl.program_id(0)

  @pl.when(t == 0)
  def _():
    for r in (hf, cf, hb, cb):
      r[...] = jnp.zeros_like(r)

  h2f, c2f = _lstm_step([ff_ref[...], fb_ref[...]], wf, bsf, hf, cf)
  hf[...] = h2f
  cf[...] = c2f

  h2b, c2b = _lstm_step([rf_ref[...], rb_ref[...]], wb, bsb, hb, cb)
  hb[...] = h2b
  cb[...] = c2b

  @pl.when(t == 0)
  def _():
    yb_last[...] = h2b

  @pl.when(t == T - 1)
  def _():
    logits = (jnp.dot(h2f, fcw[:H, :], preferred_element_type=jnp.float32)
              + jnp.dot(yb_last[...], fcw[H:, :],
                        preferred_element_type=jnp.float32)
              + fcb[...])
    out_ref[...] = logits


def _rep(shape):
  return pl.BlockSpec(shape, lambda t: tuple(0 for _ in shape))


def _bilstm_l0(x, wf, bsf, wb, bsb):
  fwd = pl.BlockSpec((B, DP), lambda t: (t, 0))
  rev = pl.BlockSpec((B, DP), lambda t: (T - 1 - t, 0))
  return pl.pallas_call(
      _layer0_body,
      grid=(T,),
      in_specs=[fwd, rev, _rep((DP + H, G4)), _rep((1, G4)),
                _rep((DP + H, G4)), _rep((1, G4))],
      out_specs=[pl.BlockSpec((B, H), lambda t: (t, 0)),
                 pl.BlockSpec((B, H), lambda t: (T - 1 - t, 0))],
      out_shape=[jax.ShapeDtypeStruct((BT, H), jnp.bfloat16)] * 2,
      scratch_shapes=[pltpu.VMEM((B, H), jnp.float32)] * 4,
  )(x, x, wf, bsf, wb, bsb)


def _bilstm_l1_fc(yf, yb, wf, bsf, wb, bsb, fcw_t, fcb):
  fwd = pl.BlockSpec((B, H), lambda t: (t, 0))
  rev = pl.BlockSpec((B, H), lambda t: (T - 1 - t, 0))
  return pl.pallas_call(
      _layer1_body,
      grid=(T,),
      in_specs=[fwd, fwd, rev, rev,
                _rep((3 * H, G4)), _rep((1, G4)),
                _rep((3 * H, G4)), _rep((1, G4)),
                _rep((2 * H, NC)), _rep((1, NC))],
      out_specs=pl.BlockSpec((B, NC), lambda t: (0, 0)),
      out_shape=jax.ShapeDtypeStruct((B, NC), jnp.float32),
      scratch_shapes=[pltpu.VMEM((B, H), jnp.float32)] * 5,
  )(yf, yb, yf, yb, wf, bsf, wb, bsb, fcw_t, fcb)


def kernel(indices, emb_table,
           W_ih_l0_f, W_hh_l0_f, b_ih_l0_f, b_hh_l0_f,
           W_ih_l0_b, W_hh_l0_b, b_ih_l0_b, b_hh_l0_b,
           W_ih_l1_f, W_hh_l1_f, b_ih_l1_f, b_hh_l1_f,
           W_ih_l1_b, W_hh_l1_b, b_ih_l1_b, b_hh_l1_b,
           fc_W, fc_b):
  idx_flat = indices.T.reshape(BT)
  table_p = _pad_table(emb_table.T)
  x = _sc_gather(table_p, idx_flat)

  def prep(wih, whh, bih, bhh, pad=0):
    wt = wih.T
    if pad:
      wt = jnp.pad(wt, ((0, pad), (0, 0)))
    wcat = jnp.concatenate([wt, whh.T], axis=0)
    scale = jnp.concatenate([jnp.full((H,), 0.5, jnp.float32)] * 2
                            + [jnp.ones((H,), jnp.float32),
                               jnp.full((H,), 0.5, jnp.float32)])
    return ((wcat * scale[None, :]).astype(jnp.bfloat16),
            ((bih + bhh) * scale).reshape(1, G4))

  w0f = prep(W_ih_l0_f, W_hh_l0_f, b_ih_l0_f, b_hh_l0_f, DP - D)
  w0b = prep(W_ih_l0_b, W_hh_l0_b, b_ih_l0_b, b_hh_l0_b, DP - D)
  yf, ybk = _bilstm_l0(x, *w0f, *w0b)

  w1f = prep(W_ih_l1_f, W_hh_l1_f, b_ih_l1_f, b_hh_l1_f)
  w1b = prep(W_ih_l1_b, W_hh_l1_b, b_ih_l1_b, b_hh_l1_b)
  return _bilstm_l1_fc(yf, ybk, *w1f, *w1b, fc_W.T, fc_b.reshape(1, NC))


# speedup vs baseline: 1.5458x; 1.0042x over previous
import functools

import jax
import jax.numpy as jnp
from jax import lax
from jax.experimental import pallas as pl
from jax.experimental.pallas import tpu as pltpu
from jax.experimental.pallas import tpu_sc as plsc

B, T, V, D, H, NC = 1024, 50, 100000, 100, 128, 7
DP = 128
G4 = 4 * H
BT = B * T

_NCORE, _NSUB = 2, 16
_NW = _NCORE * _NSUB
_PER_W = BT // _NW
_CHUNK = 80
_NCHUNK = _PER_W // _CHUNK


def _sc_gather(table, idx_flat):
  mesh = plsc.VectorSubcoreMesh(core_axis_name="c", subcore_axis_name="s")

  @functools.partial(
      pl.kernel,
      out_type=jax.ShapeDtypeStruct((BT, DP), jnp.float32),
      mesh=mesh,
      scratch_types=[
          pltpu.VMEM((_PER_W,), jnp.int32),
          pltpu.VMEM((_CHUNK, DP), jnp.float32),
          pltpu.VMEM((_CHUNK, DP), jnp.float32),
          pltpu.SemaphoreType.DMA,
          pltpu.SemaphoreType.DMA,
      ],
  )
  def gather_kernel(table_hbm, idx_hbm, out_hbm, idx_v, rows_a, rows_b, sem_a,
                    sem_b):
    wid = lax.axis_index("s") * _NCORE + lax.axis_index("c")
    base = wid * _PER_W
    pltpu.sync_copy(idx_hbm.at[pl.ds(base, _PER_W)], idx_v)
    bufs = ((rows_a, sem_a), (rows_b, sem_b))

    def start(j):
      rows, sem = bufs[j % 2]
      return pltpu.async_copy(
          table_hbm.at[idx_v.at[pl.ds(j * _CHUNK, _CHUNK)]], rows, sem)

    pending = start(0)
    for j in range(_NCHUNK):
      nxt = start(j + 1) if j + 1 < _NCHUNK else None
      pending.wait()
      pltpu.sync_copy(bufs[j % 2][0],
                      out_hbm.at[pl.ds(base + j * _CHUNK, _CHUNK)])
      pending = nxt

  return gather_kernel(table, idx_flat)


def _tpad_body(xt_ref, o_ref):
  blk = o_ref.shape[0]
  rows = xt_ref[...].T
  o_ref[...] = jnp.concatenate(
      [rows, jnp.zeros((blk, DP - D), jnp.float32)], axis=1)


def _pad_table(table_t):
  blkc = 16384
  nblk = (V + blkc - 1) // blkc
  return pl.pallas_call(
      _tpad_body,
      grid=(nblk,),
      in_specs=[pl.BlockSpec((D, blkc), lambda i: (0, i))],
      out_specs=pl.BlockSpec((blkc, DP), lambda i: (i, 0)),
      out_shape=jax.ShapeDtypeStruct((V, DP), jnp.float32),
  )(table_t)


def _lstm_step(x_parts, w_t, bias, h, c):
  xin = jnp.concatenate(
      [xp.astype(jnp.bfloat16) for xp in x_parts] + [h[...]], axis=1)
  g = jnp.dot(xin, w_t[...], preferred_element_type=jnp.float32) + bias[...]
  gi = 0.5 * jnp.tanh(g[:, :H]) + 0.5
  gf = 0.5 * jnp.tanh(g[:, H:2 * H]) + 0.5
  gg = jnp.tanh(g[:, 2 * H:3 * H])
  go = 0.5 * jnp.tanh(g[:, 3 * H:]) + 0.5
  c2 = gf * c[...] + gi * gg
  h2 = go * jnp.tanh(c2)
  return h2, c2


def _layer0_body(xf_ref, xb_ref, wf, bsf, wb, bsb,
                 yf_ref, yb_ref, hf, cf, hb, cb):
  t = pl.program_id(0)

  @pl.when(t == 0)
  def _():
    for r in (hf, cf, hb, cb):
      r[...] = jnp.zeros_like(r)

  h2f, c2f = _lstm_step([xf_ref[...]], wf, bsf, hf, cf)
  h2fb = h2f.astype(jnp.bfloat16)
  hf[...] = h2fb
  cf[...] = c2f
  yf_ref[...] = h2fb

  h2b, c2b = _lstm_step([xb_ref[...]], wb, bsb, hb, cb)
  h2bb = h2b.astype(jnp.bfloat16)
  hb[...] = h2bb
  cb[...] = c2b
  yb_ref[...] = h2bb


def _layer1_body(ff_ref, fb_ref, rf_ref, rb_ref, wf, bsf, wb, bsb,
                 fcw, fcb, out_ref, hf, cf, hb, cb, yb_last):
  t = pl.program_id(0)

  @pl.when(t == 0)
  def _():
    for r in (hf, cf, hb, cb):
      r[...] = jnp.zeros_like(r)

  h2f, c2f = _lstm_step([ff_ref[...], fb_ref[...]], wf, bsf, hf, cf)
  hf[...] = h2f.astype(jnp.bfloat16)
  cf[...] = c2f

  h2b, c2b = _lstm_step([rf_ref[...], rb_ref[...]], wb, bsb, hb, cb)
  hb[...] = h2b.astype(jnp.bfloat16)
  cb[...] = c2b

  @pl.when(t == 0)
  def _():
    yb_last[...] = h2b

  @pl.when(t == T - 1)
  def _():
    logits = (jnp.dot(h2f, fcw[:H, :], preferred_element_type=jnp.float32)
              + jnp.dot(yb_last[...], fcw[H:, :],
                        preferred_element_type=jnp.float32)
              + fcb[...])
    out_ref[...] = logits


def _rep(shape):
  return pl.BlockSpec(shape, lambda t: tuple(0 for _ in shape))


def _bilstm_l0(x, wf, bsf, wb, bsb):
  fwd = pl.BlockSpec((B, DP), lambda t: (t, 0))
  rev = pl.BlockSpec((B, DP), lambda t: (T - 1 - t, 0))
  return pl.pallas_call(
      _layer0_body,
      grid=(T,),
      in_specs=[fwd, rev, _rep((DP + H, G4)), _rep((1, G4)),
                _rep((DP + H, G4)), _rep((1, G4))],
      out_specs=[pl.BlockSpec((B, H), lambda t: (t, 0)),
                 pl.BlockSpec((B, H), lambda t: (T - 1 - t, 0))],
      out_shape=[jax.ShapeDtypeStruct((BT, H), jnp.bfloat16)] * 2,
      scratch_shapes=[pltpu.VMEM((B, H), jnp.bfloat16),
                      pltpu.VMEM((B, H), jnp.float32),
                      pltpu.VMEM((B, H), jnp.bfloat16),
                      pltpu.VMEM((B, H), jnp.float32)],
  )(x, x, wf, bsf, wb, bsb)


def _bilstm_l1_fc(yf, yb, wf, bsf, wb, bsb, fcw_t, fcb):
  fwd = pl.BlockSpec((B, H), lambda t: (t, 0))
  rev = pl.BlockSpec((B, H), lambda t: (T - 1 - t, 0))
  return pl.pallas_call(
      _layer1_body,
      grid=(T,),
      in_specs=[fwd, fwd, rev, rev,
                _rep((3 * H, G4)), _rep((1, G4)),
                _rep((3 * H, G4)), _rep((1, G4)),
                _rep((2 * H, NC)), _rep((1, NC))],
      out_specs=pl.BlockSpec((B, NC), lambda t: (0, 0)),
      out_shape=jax.ShapeDtypeStruct((B, NC), jnp.float32),
      scratch_shapes=[pltpu.VMEM((B, H), jnp.bfloat16),
                      pltpu.VMEM((B, H), jnp.float32),
                      pltpu.VMEM((B, H), jnp.bfloat16),
                      pltpu.VMEM((B, H), jnp.float32),
                      pltpu.VMEM((B, H), jnp.float32)],
  )(yf, yb, yf, yb, wf, bsf, wb, bsb, fcw_t, fcb)


def kernel(indices, emb_table,
           W_ih_l0_f, W_hh_l0_f, b_ih_l0_f, b_hh_l0_f,
           W_ih_l0_b, W_hh_l0_b, b_ih_l0_b, b_hh_l0_b,
           W_ih_l1_f, W_hh_l1_f, b_ih_l1_f, b_hh_l1_f,
           W_ih_l1_b, W_hh_l1_b, b_ih_l1_b, b_hh_l1_b,
           fc_W, fc_b):
  idx_flat = indices.T.reshape(BT)
  table_p = _pad_table(emb_table.T)
  x = _sc_gather(table_p, idx_flat)

  def prep(wih, whh, bih, bhh, pad=0):
    wt = wih.T
    if pad:
      wt = jnp.pad(wt, ((0, pad), (0, 0)))
    wcat = jnp.concatenate([wt, whh.T], axis=0)
    scale = jnp.concatenate([jnp.full((H,), 0.5, jnp.float32)] * 2
                            + [jnp.ones((H,), jnp.float32),
                               jnp.full((H,), 0.5, jnp.float32)])
    return ((wcat * scale[None, :]).astype(jnp.bfloat16),
            ((bih + bhh) * scale).reshape(1, G4))

  w0f = prep(W_ih_l0_f, W_hh_l0_f, b_ih_l0_f, b_hh_l0_f, DP - D)
  w0b = prep(W_ih_l0_b, W_hh_l0_b, b_ih_l0_b, b_hh_l0_b, DP - D)
  yf, ybk = _bilstm_l0(x, *w0f, *w0b)

  w1f = prep(W_ih_l1_f, W_hh_l1_f, b_ih_l1_f, b_hh_l1_f)
  w1b = prep(W_ih_l1_b, W_hh_l1_b, b_ih_l1_b, b_hh_l1_b)
  return _bilstm_l1_fc(yf, ybk, *w1f, *w1b, fc_W.T, fc_b.reshape(1, NC))
